# Initial kernel scaffold; baseline (speedup 1.0000x reference)
import jax
import jax.numpy as jnp
from jax.experimental import pallas as pl


def kernel(x, edge_index, edge_type, edge_weight, W_bases, w_comp):
    raise NotImplementedError("write your pallas kernel here")


# speedup vs baseline: 13.4688x; 13.4688x over previous
import functools

import jax
import jax.numpy as jnp
from jax import lax
from jax.experimental import pallas as pl
from jax.experimental.pallas import tpu as pltpu
from jax.experimental.pallas import tpu_sc as plsc

N_NODES = 10000
N_EDGES = 320000
IN_FEAT = 128
OUT_FEAT = 128
NUM_BASES = 4
NUM_RELS = 8

NC = 2
NS = 16
NW = NC * NS
B = 128
NBLK = 79
EPW = NBLK * B
E_PAD = NW * EPW
ROWS_PER_TILE = 632
N_PAD = ROWS_PER_TILE * NS
L = 16


def _xw_body(x_ref, weff_ref, out_ref):
    out_ref[...] = jnp.dot(x_ref[...], weff_ref[0],
                           preferred_element_type=jnp.float32)


def _tc_transform(x, weff):
    BN = 1000
    grid = (NUM_RELS, N_NODES // BN)
    return pl.pallas_call(
        _xw_body,
        grid=grid,
        in_specs=[
            pl.BlockSpec((BN, IN_FEAT), lambda r, nb: (nb, 0)),
            pl.BlockSpec((1, IN_FEAT, OUT_FEAT), lambda r, nb: (r, 0, 0)),
        ],
        out_specs=pl.BlockSpec(
            (BN, OUT_FEAT),
            lambda r, nb: (r * (N_NODES // BN) + nb, 0)),
        out_shape=jax.ShapeDtypeStruct((NUM_RELS * N_NODES, OUT_FEAT),
                                       jnp.float32),
    )(x, weff)


def _gidx_body(src_ref, typ_ref, out_ref):
    out_ref[...] = typ_ref[...] * N_NODES + src_ref[...]


def _tc_gidx(src, typ):
    return pl.pallas_call(
        _gidx_body,
        out_shape=jax.ShapeDtypeStruct((E_PAD // 128, 128), jnp.int32),
    )(src.reshape(E_PAD // 128, 128), typ.reshape(E_PAD // 128, 128))


def _sc_body(xw_hbm, gidx_hbm, dst_hbm, ew_hbm, zeros_hbm, out_hbm,
             gidx_v, dst_v, ew_v, rows_v, accum):
    c = lax.axis_index("c")
    s = lax.axis_index("s")
    wid = s * NC + c

    row0 = s * ROWS_PER_TILE
    pltpu.sync_copy(zeros_hbm, accum.at[pl.ds(row0, ROWS_PER_TILE)])

    pltpu.sync_copy(gidx_hbm.at[wid], gidx_v)
    pltpu.sync_copy(dst_hbm.at[wid], dst_v)
    pltpu.sync_copy(ew_hbm.at[wid], ew_v)

    plsc.subcore_barrier()

    def block(k, carry):
        pltpu.sync_copy(xw_hbm.at[gidx_v.at[k]], rows_v)

        for g in range(B // L):
            ewv = ew_v[k, pl.ds(g * L, L)]
            for i16 in range(L):
                ew_b = lax.gather(
                    ewv, jnp.full((L, 1), i16, jnp.int32),
                    lax.GatherDimensionNumbers(
                        offset_dims=(), collapsed_slice_dims=(0,),
                        start_index_map=(0,)),
                    slice_sizes=(1,),
                    mode=lax.GatherScatterMode.PROMISE_IN_BOUNDS)
                i = g * L + i16
                for j in range(IN_FEAT // L):
                    sl = pl.ds(j * L, L)
                    rows_v[i, sl] = rows_v[i, sl] * ew_b

        pltpu.sync_copy(rows_v, accum.at[dst_v.at[k]], add=True)
        return carry

    lax.fori_loop(0, NBLK, block, 0)

    plsc.subcore_barrier()

    pltpu.sync_copy(accum.at[pl.ds(row0, ROWS_PER_TILE)],
                    out_hbm.at[c, pl.ds(row0, ROWS_PER_TILE)])


def _sc_scatter(xw, gidx, dst, ew, zeros):
    mesh = plsc.VectorSubcoreMesh(core_axis_name="c", subcore_axis_name="s",
                                  num_cores=NC, num_subcores=NS)
    f = pl.kernel(
        _sc_body,
        out_type=jax.ShapeDtypeStruct((NC, N_PAD, OUT_FEAT), jnp.float32),
        mesh=mesh,
        scratch_types=[
            pltpu.VMEM((NBLK, B), jnp.int32),
            pltpu.VMEM((NBLK, B), jnp.int32),
            pltpu.VMEM((NBLK, B), jnp.float32),
            pltpu.VMEM((B, IN_FEAT), jnp.float32),
            pltpu.VMEM_SHARED((N_PAD, OUT_FEAT), jnp.float32),
        ],
    )
    return f(xw, gidx, dst, ew, zeros)


def _add_body(p_ref, out_ref):
    out_ref[...] = p_ref[0] + p_ref[1]


def _tc_add(partial):
    BN = 1000
    return pl.pallas_call(
        _add_body,
        grid=(N_NODES // BN,),
        in_specs=[pl.BlockSpec((NC, BN, OUT_FEAT), lambda nb: (0, nb, 0))],
        out_shape=jax.ShapeDtypeStruct((N_NODES, OUT_FEAT), jnp.float32),
        out_specs=pl.BlockSpec((BN, OUT_FEAT), lambda nb: (nb, 0)),
    )(partial)


def kernel(x, edge_index, edge_type, edge_weight, W_bases, w_comp):
    npad = E_PAD - N_EDGES
    src = jnp.pad(edge_index[0].astype(jnp.int32), (0, npad))
    typ = jnp.pad(edge_type.astype(jnp.int32), (0, npad))
    dst = jnp.pad(edge_index[1].astype(jnp.int32), (0, npad),
                  constant_values=N_NODES).reshape(NW, NBLK, B)
    ew = jnp.pad(edge_weight.astype(jnp.float32),
                 (0, npad)).reshape(NW, NBLK, B)
    zeros = jnp.zeros((ROWS_PER_TILE, OUT_FEAT), jnp.float32)

    composed = jnp.matmul(w_comp, jnp.transpose(W_bases, (1, 0, 2)))
    weff = composed.reshape(NUM_RELS, IN_FEAT, OUT_FEAT)

    gidx = _tc_gidx(src, typ).reshape(NW, NBLK, B)
    xw = _tc_transform(x, weff)
    partial = _sc_scatter(xw, gidx, dst, ew, zeros)
    return _tc_add(partial)
